# baseline (speedup 1.0000x reference)
import functools

import jax
import jax.numpy as jnp
from jax import lax
from jax.experimental import pallas as pl
from jax.experimental.pallas import tpu as pltpu
from jax.experimental.pallas import tpu_sc as plsc

_N = 10000
_E = 320000
_H = 128
_G = 64

_NC = 2
_NS = 16
_NW = _NC * _NS
_K = 128
_NCH0 = 128
_NCH1 = 32
_NCHT = _NS * (_NCH0 + _NCH1)
_EPAD = _NCHT * _K
_NPAD = 10112
_RPT = _NPAD // _NS
_TRASH = _NPAD - 1


def _agg_body(h_hbm, src16_hbm, dst16_hbm, out_hbm,
              rows_a, rows_b, src16_v, dst16_v, sidx_a, sidx_b, didx,
              acc_sh, sem_ga, sem_gb):
    cid = lax.axis_index("c")
    sid = lax.axis_index("s")

    def zstore(i, carry):
        for j in range(_H // 16):
            rows_a[i, pl.ds(j * 16, 16)] = jnp.zeros((16,), jnp.float32)
        return carry
    lax.fori_loop(0, _K, zstore, 0)
    def zcopy(j, carry):
        pltpu.sync_copy(rows_a, acc_sh.at[pl.ds(sid * _RPT + j * _K, _K)])
        return carry
    lax.fori_loop(0, _RPT // _K, zcopy, 0)
    rem = _RPT % _K
    pltpu.sync_copy(rows_a.at[pl.ds(0, rem)],
                    acc_sh.at[pl.ds(sid * _RPT + (_RPT // _K) * _K, rem)])
    plsc.subcore_barrier()

    def g_start(sidx, buf, sem):
        pltpu.async_copy(h_hbm.at[sidx], buf, sem)

    def g_wait(sidx, buf, sem):
        pltpu.make_async_copy(h_hbm.at[sidx], buf, sem).wait()

    def s_add(dbuf, buf):
        pltpu.sync_copy(buf, acc_sh.at[dbuf], add=True)

    def unpack(pk_v, lc, parity, out):
        row = lc // 2
        col = parity * (_K // 2)
        for k in range(_K // 32):
            w = pk_v[row, pl.ds(col + 16 * k, 16)]
            out[pl.ds(32 * k, 16)] = jnp.bitwise_and(w, 0xFFFF)
            out[pl.ds(32 * k + 16, 16)] = lax.shift_right_logical(w, 16)

    def pipeline(base2, nch):
        pltpu.sync_copy(src16_hbm.at[pl.ds(base2, nch // 2)],
                        src16_v.at[pl.ds(0, nch // 2)])
        pltpu.sync_copy(dst16_hbm.at[pl.ds(base2, nch // 2)],
                        dst16_v.at[pl.ds(0, nch // 2)])
        unpack(src16_v, 0, 0, sidx_a)
        g_start(sidx_a, rows_a, sem_ga)
        unpack(src16_v, 1, 1, sidx_b)
        g_start(sidx_b, rows_b, sem_gb)

        def body(t, carry):
            lc = 2 * t
            g_wait(sidx_a, rows_a, sem_ga)
            unpack(dst16_v, lc, 0, didx)
            s_add(didx, rows_a)
            unpack(src16_v, lc + 2, 0, sidx_a)
            g_start(sidx_a, rows_a, sem_ga)
            g_wait(sidx_b, rows_b, sem_gb)
            unpack(dst16_v, lc + 1, 1, didx)
            s_add(didx, rows_b)
            unpack(src16_v, lc + 3, 1, sidx_b)
            g_start(sidx_b, rows_b, sem_gb)
            return carry

        lax.fori_loop(0, nch // 2 - 1, body, 0)
        lc = nch - 2
        g_wait(sidx_a, rows_a, sem_ga)
        unpack(dst16_v, lc, 0, didx)
        s_add(didx, rows_a)
        g_wait(sidx_b, rows_b, sem_gb)
        unpack(dst16_v, lc + 1, 1, didx)
        s_add(didx, rows_b)

    @pl.when(cid == 0)
    def _():
        pipeline(sid * (_NCH0 // 2), _NCH0)

    @pl.when(cid == 1)
    def _():
        pipeline(_NS * (_NCH0 // 2) + sid * (_NCH1 // 2), _NCH1)

    plsc.subcore_barrier()
    pltpu.sync_copy(acc_sh.at[pl.ds(sid * _RPT, _RPT)],
                    out_hbm.at[cid].at[pl.ds(sid * _RPT, _RPT)])


@functools.cache
def _make_agg():
    mesh = plsc.VectorSubcoreMesh(core_axis_name="c", subcore_axis_name="s",
                                  num_cores=_NC, num_subcores=_NS)
    return pl.kernel(
        _agg_body,
        out_type=jax.ShapeDtypeStruct((_NC, _NPAD, _H), jnp.float32),
        mesh=mesh,
        scratch_types=[
            pltpu.VMEM((_K, _H), jnp.float32),
            pltpu.VMEM((_K, _H), jnp.float32),
            pltpu.VMEM((_K // 2, _K), jnp.int32),
            pltpu.VMEM((_K // 2, _K), jnp.int32),
            pltpu.VMEM((_K,), jnp.int32),
            pltpu.VMEM((_K,), jnp.int32),
            pltpu.VMEM((_K,), jnp.int32),
            pltpu.VMEM_SHARED((_NPAD, _H), jnp.float32),
            pltpu.SemaphoreType.DMA,
            pltpu.SemaphoreType.DMA,
        ],
    )


_ROWS = 2000
_NBLK = _N // _ROWS


def _mlp_body(h_ref, a_ref, bid_ref, w1_ref, b1_ref, w2_ref, b2_ref,
              hout_ref, pool_ref):
    z = h_ref[...] + a_ref[0] + a_ref[1]
    z = jnp.dot(z, w1_ref[...], preferred_element_type=jnp.float32) + b1_ref[...]
    z = jnp.maximum(z, 0.0)
    z = jnp.dot(z, w2_ref[...], preferred_element_type=jnp.float32) + b2_ref[...]
    h2 = jnp.maximum(z, 0.0)
    hout_ref[...] = h2

    bid = bid_ref[0]
    onehot = (bid == lax.broadcasted_iota(jnp.int32, (_G, _ROWS), 0))
    contrib = jnp.dot(onehot.astype(jnp.float32), h2,
                      preferred_element_type=jnp.float32)

    @pl.when(pl.program_id(0) == 0)
    def _():
        pool_ref[...] = jnp.zeros_like(pool_ref)

    pool_ref[...] += contrib


_mlp = pl.pallas_call(
    _mlp_body,
    grid=(_NBLK,),
    in_specs=[
        pl.BlockSpec((_ROWS, _H), lambda i: (i, 0)),
        pl.BlockSpec((_NC, _ROWS, _H), lambda i: (0, i, 0)),
        pl.BlockSpec((1, 1, _ROWS), lambda i: (i, 0, 0)),
        pl.BlockSpec((_H, _H), lambda i: (0, 0)),
        pl.BlockSpec((1, _H), lambda i: (0, 0)),
        pl.BlockSpec((_H, _H), lambda i: (0, 0)),
        pl.BlockSpec((1, _H), lambda i: (0, 0)),
    ],
    out_specs=[
        pl.BlockSpec((_ROWS, _H), lambda i: (i, 0)),
        pl.BlockSpec((_G, _H), lambda i: (0, 0)),
    ],
    out_shape=[
        jax.ShapeDtypeStruct((_N, _H), jnp.float32),
        jax.ShapeDtypeStruct((_G, _H), jnp.float32),
    ],
)


def _ffn_body(p0_ref, p1_ref, p2_ref, wf1_ref, bf1_ref, wf2_ref, bf2_ref,
              out_ref):
    gr = jnp.concatenate([p0_ref[...], p1_ref[...], p2_ref[...]], axis=1)
    z = jnp.dot(gr, wf1_ref[...], preferred_element_type=jnp.float32) + bf1_ref[...]
    z = jnp.maximum(z, 0.0)
    out_ref[...] = (jnp.dot(z, wf2_ref[...], preferred_element_type=jnp.float32)
                    + bf2_ref[...])


_ffn = pl.pallas_call(
    _ffn_body,
    out_shape=jax.ShapeDtypeStruct((_G, _G), jnp.float32),
)


@jax.jit
def kernel(x, edge_index, batch,
           W1_0, b1_0, W2_0, b2_0,
           W1_1, b1_1, W2_1, b2_1,
           W1_2, b1_2, W2_2, b2_2,
           Wf1, bf1, Wf2, bf2):
    pad = _EPAD - _E
    srcp = jnp.concatenate(
        [edge_index[0], jnp.zeros((pad,), jnp.int32)]).reshape(_NCHT, 4, 2, 16)
    src16 = (srcp[:, :, 0, :] | (srcp[:, :, 1, :] << 16)).reshape(_NCHT // 2, _K)
    dstp = jnp.concatenate(
        [edge_index[1], jnp.full((pad,), _TRASH, jnp.int32)]).reshape(_NCHT, 4, 2, 16)
    dst16 = (dstp[:, :, 0, :] | (dstp[:, :, 1, :] << 16)).reshape(_NCHT // 2, _K)
    bid3 = batch.reshape(_NBLK, 1, _ROWS)

    layers = [(W1_0, b1_0, W2_0, b2_0),
              (W1_1, b1_1, W2_1, b2_1),
              (W1_2, b1_2, W2_2, b2_2)]
    agg_fn = _make_agg()
    h = x
    pools = []
    for (w1, b1, w2, b2) in layers:
        agg = agg_fn(h, src16, dst16)
        h, pool = _mlp(h, agg, bid3, w1, b1.reshape(1, _H),
                       w2, b2.reshape(1, _H))
        pools.append(pool)

    return _ffn(pools[0], pools[1], pools[2],
                Wf1, bf1.reshape(1, -1), Wf2, bf2.reshape(1, -1))


# speedup vs baseline: 1.1253x; 1.1253x over previous
import functools

import jax
import jax.numpy as jnp
from jax import lax
from jax.experimental import pallas as pl
from jax.experimental.pallas import tpu as pltpu
from jax.experimental.pallas import tpu_sc as plsc

_N = 10000
_E = 320000
_H = 128
_G = 64

_NC = 2
_NS = 16
_NW = _NC * _NS
_K = 128
_NCH0 = 128
_NCH1 = 32
_NCHT = _NS * (_NCH0 + _NCH1)
_EPAD = _NCHT * _K
_NPAD = 10240
_RPT = _NPAD // _NS
_TRASH = _NPAD - 1


def _agg_body(h_hbm, src_hbm, dst_hbm, out_hbm,
              rows_a, rows_b, sidx_a, sidx_b, didx_a, didx_b,
              acc_sh, sem_ia, sem_ib, sem_ga, sem_gb):
    cid = lax.axis_index("c")
    sid = lax.axis_index("s")

    def zstore(i, carry):
        for j in range(_H // 16):
            rows_a[i, pl.ds(j * 16, 16)] = jnp.zeros((16,), jnp.float32)
        return carry
    lax.fori_loop(0, _K, zstore, 0)
    def zcopy(j, carry):
        pltpu.sync_copy(rows_a, acc_sh.at[pl.ds(sid * _RPT + j * _K, _K)])
        return carry
    lax.fori_loop(0, _RPT // _K, zcopy, 0)
    plsc.subcore_barrier()

    def i_start(c, idx_hbm, buf, sem):
        pltpu.async_copy(idx_hbm.at[c], buf, sem)

    def i_wait(c, idx_hbm, buf, sem):
        pltpu.make_async_copy(idx_hbm.at[c], buf, sem).wait()

    def g_start(sidx, buf, sem):
        pltpu.async_copy(h_hbm.at[sidx], buf, sem)

    def g_wait(sidx, buf, sem):
        pltpu.make_async_copy(h_hbm.at[sidx], buf, sem).wait()

    def s_add(didx, buf):
        pltpu.sync_copy(buf, acc_sh.at[didx], add=True)

    def fill(c, sidx, didx, sem_i, sem_g, buf):
        i_start(c, src_hbm, sidx, sem_i)
        i_start(c, dst_hbm, didx, sem_i)
        i_wait(c, src_hbm, sidx, sem_i)
        i_wait(c, dst_hbm, didx, sem_i)
        g_start(sidx, buf, sem_g)

    def pipeline(base, nch):
        fill(base, sidx_a, didx_a, sem_ia, sem_ga, rows_a)
        fill(base + 1, sidx_b, didx_b, sem_ib, sem_gb, rows_b)

        def body(t, carry):
            c = base + 2 * t
            g_wait(sidx_a, rows_a, sem_ga)
            s_add(didx_a, rows_a)
            fill(c + 2, sidx_a, didx_a, sem_ia, sem_ga, rows_a)
            g_wait(sidx_b, rows_b, sem_gb)
            s_add(didx_b, rows_b)
            fill(c + 3, sidx_b, didx_b, sem_ib, sem_gb, rows_b)
            return carry

        lax.fori_loop(0, nch // 2 - 1, body, 0)
        g_wait(sidx_a, rows_a, sem_ga)
        s_add(didx_a, rows_a)
        g_wait(sidx_b, rows_b, sem_gb)
        s_add(didx_b, rows_b)

    @pl.when(cid == 0)
    def _():
        pipeline(sid * _NCH0, _NCH0)

    @pl.when(cid == 1)
    def _():
        pipeline(_NS * _NCH0 + sid * _NCH1, _NCH1)

    plsc.subcore_barrier()
    pltpu.sync_copy(acc_sh.at[pl.ds(sid * _RPT, _RPT)],
                    out_hbm.at[cid].at[pl.ds(sid * _RPT, _RPT)])


@functools.cache
def _make_agg():
    mesh = plsc.VectorSubcoreMesh(core_axis_name="c", subcore_axis_name="s",
                                  num_cores=_NC, num_subcores=_NS)
    return pl.kernel(
        _agg_body,
        out_type=jax.ShapeDtypeStruct((_NC, _NPAD, _H), jnp.float32),
        mesh=mesh,
        scratch_types=[
            pltpu.VMEM((_K, _H), jnp.float32),
            pltpu.VMEM((_K, _H), jnp.float32),
            pltpu.VMEM((_K,), jnp.int32),
            pltpu.VMEM((_K,), jnp.int32),
            pltpu.VMEM((_K,), jnp.int32),
            pltpu.VMEM((_K,), jnp.int32),
            pltpu.VMEM_SHARED((_NPAD, _H), jnp.float32),
            pltpu.SemaphoreType.DMA,
            pltpu.SemaphoreType.DMA,
            pltpu.SemaphoreType.DMA,
            pltpu.SemaphoreType.DMA,
        ],
    )


_ROWS = 2000
_NBLK = _N // _ROWS


def _mlp_body(h_ref, a_ref, bid_ref, w1_ref, b1_ref, w2_ref, b2_ref,
              hout_ref, pool_ref):
    z = h_ref[...] + a_ref[0] + a_ref[1]
    z = jnp.dot(z, w1_ref[...], preferred_element_type=jnp.float32) + b1_ref[...]
    z = jnp.maximum(z, 0.0)
    z = jnp.dot(z, w2_ref[...], preferred_element_type=jnp.float32) + b2_ref[...]
    h2 = jnp.maximum(z, 0.0)
    hout_ref[...] = h2

    bid = bid_ref[0]
    onehot = (bid == lax.broadcasted_iota(jnp.int32, (_G, _ROWS), 0))
    contrib = jnp.dot(onehot.astype(jnp.float32), h2,
                      preferred_element_type=jnp.float32)

    @pl.when(pl.program_id(0) == 0)
    def _():
        pool_ref[...] = jnp.zeros_like(pool_ref)

    pool_ref[...] += contrib


_mlp = pl.pallas_call(
    _mlp_body,
    grid=(_NBLK,),
    in_specs=[
        pl.BlockSpec((_ROWS, _H), lambda i: (i, 0)),
        pl.BlockSpec((_NC, _ROWS, _H), lambda i: (0, i, 0)),
        pl.BlockSpec((1, 1, _ROWS), lambda i: (i, 0, 0)),
        pl.BlockSpec((_H, _H), lambda i: (0, 0)),
        pl.BlockSpec((1, _H), lambda i: (0, 0)),
        pl.BlockSpec((_H, _H), lambda i: (0, 0)),
        pl.BlockSpec((1, _H), lambda i: (0, 0)),
    ],
    out_specs=[
        pl.BlockSpec((_ROWS, _H), lambda i: (i, 0)),
        pl.BlockSpec((_G, _H), lambda i: (0, 0)),
    ],
    out_shape=[
        jax.ShapeDtypeStruct((_N, _H), jnp.float32),
        jax.ShapeDtypeStruct((_G, _H), jnp.float32),
    ],
)


def _ffn_body(p0_ref, p1_ref, p2_ref, wf1_ref, bf1_ref, wf2_ref, bf2_ref,
              out_ref):
    gr = jnp.concatenate([p0_ref[...], p1_ref[...], p2_ref[...]], axis=1)
    z = jnp.dot(gr, wf1_ref[...], preferred_element_type=jnp.float32) + bf1_ref[...]
    z = jnp.maximum(z, 0.0)
    out_ref[...] = (jnp.dot(z, wf2_ref[...], preferred_element_type=jnp.float32)
                    + bf2_ref[...])


_ffn = pl.pallas_call(
    _ffn_body,
    out_shape=jax.ShapeDtypeStruct((_G, _G), jnp.float32),
)


@jax.jit
def kernel(x, edge_index, batch,
           W1_0, b1_0, W2_0, b2_0,
           W1_1, b1_1, W2_1, b2_1,
           W1_2, b1_2, W2_2, b2_2,
           Wf1, bf1, Wf2, bf2):
    pad = _EPAD - _E
    src3 = jnp.concatenate(
        [edge_index[0], jnp.zeros((pad,), jnp.int32)]).reshape(_NCHT, _K)
    dst3 = jnp.concatenate(
        [edge_index[1], jnp.full((pad,), _TRASH, jnp.int32)]).reshape(_NCHT, _K)
    bid3 = batch.reshape(_NBLK, 1, _ROWS)

    layers = [(W1_0, b1_0, W2_0, b2_0),
              (W1_1, b1_1, W2_1, b2_1),
              (W1_2, b1_2, W2_2, b2_2)]
    agg_fn = _make_agg()
    h = x
    pools = []
    for (w1, b1, w2, b2) in layers:
        agg = agg_fn(h, src3, dst3)
        h, pool = _mlp(h, agg, bid3, w1, b1.reshape(1, _H),
                       w2, b2.reshape(1, _H))
        pools.append(pool)

    return _ffn(pools[0], pools[1], pools[2],
                Wf1, bf1.reshape(1, -1), Wf2, bf2.reshape(1, -1))
